# baseline (speedup 1.0000x reference)
import jax
import jax.numpy as jnp
from jax.experimental import pallas as pl
from jax.experimental.pallas import tpu as pltpu

_N = 100000
_D = 512
_H = 256
_S = 512
_B = 1000
_W = 128
_MNEG = -1e30
_LNEG = -2e30


def _pool_kernel(firsts_ref, lasts_ref, x_ref, w1_ref, b1_ref, w2_ref,
                 seg_ref, out_ref, m_ref, d_ref):
    i = pl.program_id(0)
    nsteps = pl.num_programs(0)

    @pl.when(i == 0)
    def _init():
        out_ref[...] = jnp.zeros_like(out_ref)
        m_ref[...] = jnp.full_like(m_ref, _MNEG)
        d_ref[...] = jnp.zeros_like(d_ref)

    xb = x_ref[...]
    h = jnp.dot(xb.astype(jnp.bfloat16), w1_ref[...],
                preferred_element_type=jnp.float32)
    h = h + b1_ref[...]
    h = h * jax.nn.sigmoid(h)
    l = jnp.sum(h * w2_ref[...], axis=1)

    seg = seg_ref[0, 0, :]
    p0 = firsts_ref[i] // _W
    p1 = lasts_ref[i] // _W
    iota_w = jax.lax.broadcasted_iota(jnp.int32, (_B, _W), 1)

    for k in range(_S // _W):
        @pl.when((p0 <= k) & (k <= p1))
        def _win(k=k):
            ws = k * _W
            col = seg - ws
            onehot = col[:, None] == iota_w
            lmask = jnp.where(onehot, l[:, None], _LNEG)
            bmax = jnp.max(lmask, axis=0)

            m_old = m_ref[0, ws:ws + _W]
            m_new = jnp.maximum(m_old, bmax)
            ratio = jnp.exp(m_old - m_new)

            E = jnp.exp(lmask - m_new[None, :])

            d_ref[0, ws:ws + _W] = d_ref[0, ws:ws + _W] * ratio \
                + jnp.sum(E, axis=0)
            m_ref[0, ws:ws + _W] = m_new

            P = jax.lax.dot_general(
                E, xb, (((0,), (0,)), ((), ())),
                preferred_element_type=jnp.float32)
            out_ref[ws:ws + _W, :] = (
                out_ref[ws:ws + _W, :] * ratio[:, None] + P)

    @pl.when(i == nsteps - 1)
    def _fin():
        d = d_ref[0, :]
        out_ref[...] = out_ref[...] / (d[:, None] + 1e-16)


def kernel(x, W1, b1, W2, b2, batch):
    seg32 = batch.astype(jnp.int32)
    nblocks = _N // _B
    seg = seg32.reshape(nblocks, 1, _B)
    firsts = seg32[:: _B]
    lasts = seg32[_B - 1 :: _B]
    W1 = W1.astype(jnp.bfloat16)
    b1r = b1.reshape(1, _H)
    w2r = W2.reshape(1, _H)
    grid_spec = pltpu.PrefetchScalarGridSpec(
        num_scalar_prefetch=2,
        grid=(nblocks,),
        in_specs=[
            pl.BlockSpec((_B, _D), lambda i, f, lst: (i, 0)),
            pl.BlockSpec((_D, _H), lambda i, f, lst: (0, 0)),
            pl.BlockSpec((1, _H), lambda i, f, lst: (0, 0)),
            pl.BlockSpec((1, _H), lambda i, f, lst: (0, 0)),
            pl.BlockSpec((1, 1, _B), lambda i, f, lst: (i, 0, 0)),
        ],
        out_specs=pl.BlockSpec((_S, _D), lambda i, f, lst: (0, 0)),
        scratch_shapes=[
            pltpu.VMEM((1, _S), jnp.float32),
            pltpu.VMEM((1, _S), jnp.float32),
        ],
    )
    out = pl.pallas_call(
        _pool_kernel,
        grid_spec=grid_spec,
        out_shape=jax.ShapeDtypeStruct((_S, _D), jnp.float32),
        compiler_params=pltpu.CompilerParams(
            dimension_semantics=("arbitrary",),
        ),
    )(firsts, lasts, x, W1, b1r, w2r, seg)
    return out


# speedup vs baseline: 1.3608x; 1.3608x over previous
import jax
import jax.numpy as jnp
from jax.experimental import pallas as pl
from jax.experimental.pallas import tpu as pltpu

_N = 100000
_D = 512
_H = 256
_S = 512
_B = 2000
_W = 128
_MNEG = -1e30
_LNEG = -2e30


def _pool_kernel(firsts_ref, lasts_ref, x_ref, w1_ref, b1_ref, w2_ref,
                 seg_ref, out_ref, m_ref, d_ref):
    i = pl.program_id(0)
    nsteps = pl.num_programs(0)

    @pl.when(i == 0)
    def _init():
        out_ref[...] = jnp.zeros_like(out_ref)
        m_ref[...] = jnp.full_like(m_ref, _MNEG)
        d_ref[...] = jnp.zeros_like(d_ref)

    xb = x_ref[...]
    h = jnp.dot(xb, w1_ref[...], preferred_element_type=jnp.float32)
    h = h + b1_ref[...]
    h = h * jax.nn.sigmoid(h)
    l = jnp.sum(h * w2_ref[...], axis=1)

    seg = seg_ref[0, 0, :]
    p0 = firsts_ref[i] // _W
    p1 = lasts_ref[i] // _W
    iota_w = jax.lax.broadcasted_iota(jnp.int32, (_B, _W), 1)

    for k in range(_S // _W):
        @pl.when((p0 <= k) & (k <= p1))
        def _win(k=k):
            ws = k * _W
            col = seg - ws
            onehot = col[:, None] == iota_w
            lmask = jnp.where(onehot, l[:, None], _LNEG)
            bmax = jnp.max(lmask, axis=0)

            m_old = m_ref[0, ws:ws + _W]
            m_new = jnp.maximum(m_old, bmax)
            ratio = jnp.exp(m_old - m_new)

            E = jnp.exp(lmask - m_new[None, :])

            d_ref[0, ws:ws + _W] = d_ref[0, ws:ws + _W] * ratio \
                + jnp.sum(E, axis=0)
            m_ref[0, ws:ws + _W] = m_new

            P = jax.lax.dot_general(
                E, xb, (((0,), (0,)), ((), ())),
                preferred_element_type=jnp.float32)
            out_ref[ws:ws + _W, :] = (
                out_ref[ws:ws + _W, :] * ratio[:, None] + P)

    @pl.when(i == nsteps - 1)
    def _fin():
        d = d_ref[0, :]
        out_ref[...] = out_ref[...] / (d[:, None] + 1e-16)


def kernel(x, W1, b1, W2, b2, batch):
    seg32 = batch.astype(jnp.int32)
    nblocks = _N // _B
    seg = seg32.reshape(nblocks, 1, _B)
    firsts = seg32[:: _B]
    lasts = seg32[_B - 1 :: _B]
    b1r = b1.reshape(1, _H)
    w2r = W2.reshape(1, _H)
    grid_spec = pltpu.PrefetchScalarGridSpec(
        num_scalar_prefetch=2,
        grid=(nblocks,),
        in_specs=[
            pl.BlockSpec((_B, _D), lambda i, f, lst: (i, 0)),
            pl.BlockSpec((_D, _H), lambda i, f, lst: (0, 0)),
            pl.BlockSpec((1, _H), lambda i, f, lst: (0, 0)),
            pl.BlockSpec((1, _H), lambda i, f, lst: (0, 0)),
            pl.BlockSpec((1, 1, _B), lambda i, f, lst: (i, 0, 0)),
        ],
        out_specs=pl.BlockSpec((_S, _D), lambda i, f, lst: (0, 0)),
        scratch_shapes=[
            pltpu.VMEM((1, _S), jnp.float32),
            pltpu.VMEM((1, _S), jnp.float32),
        ],
    )
    out = pl.pallas_call(
        _pool_kernel,
        grid_spec=grid_spec,
        out_shape=jax.ShapeDtypeStruct((_S, _D), jnp.float32),
        compiler_params=pltpu.CompilerParams(
            dimension_semantics=("arbitrary",),
        ),
    )(firsts, lasts, x, W1, b1r, w2r, seg)
    return out


# speedup vs baseline: 1.5587x; 1.1454x over previous
import jax
import jax.numpy as jnp
from jax.experimental import pallas as pl
from jax.experimental.pallas import tpu as pltpu

_N = 100000
_D = 512
_H = 256
_S = 512
_B = 4000
_W = 128
_MNEG = -1e30
_LNEG = -2e30


def _pool_kernel(firsts_ref, lasts_ref, x_ref, w1_ref, b1_ref, w2_ref,
                 seg_ref, out_ref, m_ref, d_ref):
    i = pl.program_id(0)
    nsteps = pl.num_programs(0)

    @pl.when(i == 0)
    def _init():
        out_ref[...] = jnp.zeros_like(out_ref)
        m_ref[...] = jnp.full_like(m_ref, _MNEG)
        d_ref[...] = jnp.zeros_like(d_ref)

    xb = x_ref[...]
    h = jnp.dot(xb, w1_ref[...], preferred_element_type=jnp.float32)
    h = h + b1_ref[...]
    h = h * jax.nn.sigmoid(h)
    l = jnp.sum(h * w2_ref[...], axis=1)

    seg = seg_ref[0, 0, :]
    p0 = firsts_ref[i] // _W
    p1 = lasts_ref[i] // _W
    iota_w = jax.lax.broadcasted_iota(jnp.int32, (_B, _W), 1)

    for k in range(_S // _W):
        @pl.when((p0 <= k) & (k <= p1))
        def _win(k=k):
            ws = k * _W
            col = seg - ws
            onehot = col[:, None] == iota_w
            lmask = jnp.where(onehot, l[:, None], _LNEG)
            bmax = jnp.max(lmask, axis=0)

            m_old = m_ref[0, ws:ws + _W]
            m_new = jnp.maximum(m_old, bmax)
            ratio = jnp.exp(m_old - m_new)

            E = jnp.exp(lmask - m_new[None, :])

            d_ref[0, ws:ws + _W] = d_ref[0, ws:ws + _W] * ratio \
                + jnp.sum(E, axis=0)
            m_ref[0, ws:ws + _W] = m_new

            P = jax.lax.dot_general(
                E, xb, (((0,), (0,)), ((), ())),
                preferred_element_type=jnp.float32)
            out_ref[ws:ws + _W, :] = (
                out_ref[ws:ws + _W, :] * ratio[:, None] + P)

    @pl.when(i == nsteps - 1)
    def _fin():
        d = d_ref[0, :]
        out_ref[...] = out_ref[...] / (d[:, None] + 1e-16)


def kernel(x, W1, b1, W2, b2, batch):
    seg32 = batch.astype(jnp.int32)
    nblocks = _N // _B
    seg = seg32.reshape(nblocks, 1, _B)
    firsts = seg32[:: _B]
    lasts = seg32[_B - 1 :: _B]
    b1r = b1.reshape(1, _H)
    w2r = W2.reshape(1, _H)
    grid_spec = pltpu.PrefetchScalarGridSpec(
        num_scalar_prefetch=2,
        grid=(nblocks,),
        in_specs=[
            pl.BlockSpec((_B, _D), lambda i, f, lst: (i, 0)),
            pl.BlockSpec((_D, _H), lambda i, f, lst: (0, 0)),
            pl.BlockSpec((1, _H), lambda i, f, lst: (0, 0)),
            pl.BlockSpec((1, _H), lambda i, f, lst: (0, 0)),
            pl.BlockSpec((1, 1, _B), lambda i, f, lst: (i, 0, 0)),
        ],
        out_specs=pl.BlockSpec((_S, _D), lambda i, f, lst: (0, 0)),
        scratch_shapes=[
            pltpu.VMEM((1, _S), jnp.float32),
            pltpu.VMEM((1, _S), jnp.float32),
        ],
    )
    out = pl.pallas_call(
        _pool_kernel,
        grid_spec=grid_spec,
        out_shape=jax.ShapeDtypeStruct((_S, _D), jnp.float32),
        compiler_params=pltpu.CompilerParams(
            dimension_semantics=("arbitrary",),
        ),
    )(firsts, lasts, x, W1, b1r, w2r, seg)
    return out


# speedup vs baseline: 1.5663x; 1.0049x over previous
import jax
import jax.numpy as jnp
from jax.experimental import pallas as pl
from jax.experimental.pallas import tpu as pltpu

_N = 100000
_D = 512
_H = 256
_S = 512
_B = 5000
_W = 128
_MNEG = -1e30
_LNEG = -2e30


def _pool_kernel(firsts_ref, lasts_ref, x_ref, w1_ref, b1_ref, w2_ref,
                 seg_ref, out_ref, m_ref, d_ref):
    i = pl.program_id(0)
    nsteps = pl.num_programs(0)

    @pl.when(i == 0)
    def _init():
        out_ref[...] = jnp.zeros_like(out_ref)
        m_ref[...] = jnp.full_like(m_ref, _MNEG)
        d_ref[...] = jnp.zeros_like(d_ref)

    xb = x_ref[...]
    h = jnp.dot(xb, w1_ref[...], preferred_element_type=jnp.float32)
    h = h + b1_ref[...]
    h = h * jax.nn.sigmoid(h)
    l = jnp.sum(h * w2_ref[...], axis=1)

    seg = seg_ref[0, 0, :]
    p0 = firsts_ref[i] // _W
    p1 = lasts_ref[i] // _W
    iota_w = jax.lax.broadcasted_iota(jnp.int32, (_B, _W), 1)

    for k in range(_S // _W):
        @pl.when((p0 <= k) & (k <= p1))
        def _win(k=k):
            ws = k * _W
            col = seg - ws
            onehot = col[:, None] == iota_w
            lmask = jnp.where(onehot, l[:, None], _LNEG)
            bmax = jnp.max(lmask, axis=0)

            m_old = m_ref[0, ws:ws + _W]
            m_new = jnp.maximum(m_old, bmax)
            ratio = jnp.exp(m_old - m_new)

            E = jnp.exp(lmask - m_new[None, :])

            d_ref[0, ws:ws + _W] = d_ref[0, ws:ws + _W] * ratio \
                + jnp.sum(E, axis=0)
            m_ref[0, ws:ws + _W] = m_new

            P = jax.lax.dot_general(
                E, xb, (((0,), (0,)), ((), ())),
                preferred_element_type=jnp.float32)
            out_ref[ws:ws + _W, :] = (
                out_ref[ws:ws + _W, :] * ratio[:, None] + P)

    @pl.when(i == nsteps - 1)
    def _fin():
        d = d_ref[0, :]
        out_ref[...] = out_ref[...] / (d[:, None] + 1e-16)


def kernel(x, W1, b1, W2, b2, batch):
    seg32 = batch.astype(jnp.int32)
    nblocks = _N // _B
    seg = seg32.reshape(nblocks, 1, _B)
    firsts = seg32[:: _B]
    lasts = seg32[_B - 1 :: _B]
    b1r = b1.reshape(1, _H)
    w2r = W2.reshape(1, _H)
    grid_spec = pltpu.PrefetchScalarGridSpec(
        num_scalar_prefetch=2,
        grid=(nblocks,),
        in_specs=[
            pl.BlockSpec((_B, _D), lambda i, f, lst: (i, 0)),
            pl.BlockSpec((_D, _H), lambda i, f, lst: (0, 0)),
            pl.BlockSpec((1, _H), lambda i, f, lst: (0, 0)),
            pl.BlockSpec((1, _H), lambda i, f, lst: (0, 0)),
            pl.BlockSpec((1, 1, _B), lambda i, f, lst: (i, 0, 0)),
        ],
        out_specs=pl.BlockSpec((_S, _D), lambda i, f, lst: (0, 0)),
        scratch_shapes=[
            pltpu.VMEM((1, _S), jnp.float32),
            pltpu.VMEM((1, _S), jnp.float32),
        ],
    )
    out = pl.pallas_call(
        _pool_kernel,
        grid_spec=grid_spec,
        out_shape=jax.ShapeDtypeStruct((_S, _D), jnp.float32),
        compiler_params=pltpu.CompilerParams(
            dimension_semantics=("arbitrary",),
        ),
    )(firsts, lasts, x, W1, b1r, w2r, seg)
    return out
